# baseline (speedup 1.0000x reference)
import functools

import jax
import jax.numpy as jnp
from jax import lax
from jax.experimental import pallas as pl
from jax.experimental.pallas import tpu as pltpu
from jax.experimental.pallas import tpu_sc as plsc

D = 768
HID = 256
NE = 8
TOTAL = 512
TOP_K = 2
MAX_H = 3 * D
S = 2048
TS = TOP_K * S
BLK = 128
NB = (TS + NE * BLK) // BLK
P = NB * BLK
EPS = 1e-5


def _gelu(v):
    return 0.5 * v * (1.0 + lax.erf(v * 0.7071067811865476))


def _layernorm(y, g, b):
    mu = y.mean(axis=-1, keepdims=True)
    var = ((y - mu) ** 2).mean(axis=-1, keepdims=True)
    return (y - mu) / jnp.sqrt(var + EPS) * g + b


def _router_body(x_ref, t_ref, w1_ref, b1_ref, w2_ref, b2_ref, w3_ref, b3_ref,
                 tv_ref, ti_ref, loss_ref):
    x = x_ref[...]
    h = _gelu(jnp.dot(x, w1_ref[...], preferred_element_type=jnp.float32)
              + b1_ref[...])
    h = _gelu(jnp.dot(h, w2_ref[...], preferred_element_type=jnp.float32)
              + b2_ref[...])
    scores = (jnp.dot(h, w3_ref[...], preferred_element_type=jnp.float32)
              + b3_ref[...])

    p0 = jax.nn.softmax(scores, axis=-1)
    freq = p0.mean(axis=0)
    mu = freq.mean()
    loss = TOTAL * jnp.sum((freq - mu) ** 2) / (TOTAL - 1)
    loss_ref[...] = loss.reshape(1, 1)

    p = jax.nn.softmax(scores / t_ref[0, 0], axis=-1)
    iota = lax.broadcasted_iota(jnp.int32, (S, TOTAL), 1)
    m1 = p.max(axis=-1)
    i1 = jnp.where(p == m1[:, None], iota, TOTAL).min(axis=-1)
    pm = jnp.where(iota == i1[:, None], -1.0, p)
    m2 = pm.max(axis=-1)
    i2 = jnp.where(pm == m2[:, None], iota, TOTAL).min(axis=-1)
    tv_ref[0, :] = m1
    tv_ref[1, :] = m2
    ti_ref[0, :] = i1
    ti_ref[1, :] = i2


def _router(x2d, temperature, W_r1, b_r1, W_r2, b_r2, W_r3, b_r3):
    return pl.pallas_call(
        _router_body,
        out_shape=(
            jax.ShapeDtypeStruct((TOP_K, S), jnp.float32),
            jax.ShapeDtypeStruct((TOP_K, S), jnp.int32),
            jax.ShapeDtypeStruct((1, 1), jnp.float32),
        ),
    )(x2d, temperature.reshape(1, 1), W_r1, b_r1.reshape(1, HID),
      W_r2, b_r2.reshape(1, HID // 2), W_r3, b_r3.reshape(1, TOTAL))


def _pre_body(gid_ref, x_ref, w_ref, b_ref, g_ref, be_ref, o_ref):
    g = gid_ref[pl.program_id(0)]
    y = (jnp.dot(x_ref[...], w_ref[0], preferred_element_type=jnp.float32)
         + b_ref[0])
    y = _layernorm(y, g_ref[0], be_ref[0])
    r = g % 3
    y = jnp.where(r == 0, _gelu(y), jnp.where(r == 1, jax.nn.relu(y),
                                              jnp.tanh(y)))
    o_ref[...] = y


NHB = MAX_H // D


def _mlp_body(gid_ref, x_ref, w1_ref, b1_ref, w2_ref, b2_ref, o_ref):
    g = gid_ref[pl.program_id(0)]
    hb = pl.program_id(1)

    @pl.when(hb == 0)
    def _():
        o_ref[...] = jnp.broadcast_to(b2_ref[0], (BLK, D))

    @pl.when(hb < 2 + g // 4)
    def _():
        h = (jnp.dot(x_ref[...], w1_ref[0],
                     preferred_element_type=jnp.float32) + b1_ref[0])
        h = jnp.where(g % 2 == 0, _gelu(h), jax.nn.relu(h))
        o_ref[...] += jnp.dot(h, w2_ref[0],
                              preferred_element_type=jnp.float32)


def _post_body(gid_ref, x_ref, w_ref, b_ref, g_ref, be_ref, o_ref):
    g = gid_ref[pl.program_id(0)]
    z = (jnp.dot(x_ref[...], w_ref[0], preferred_element_type=jnp.float32)
         + b_ref[0])
    o_ref[...] = jnp.where(g % 2 == 0, _layernorm(z, g_ref[0], be_ref[0]), z)


def _gmm(body, x, block_gid, *weights):
    weights = tuple(w.reshape(w.shape[0], 1, w.shape[1]) if w.ndim == 2 else w
                    for w in weights)
    w_specs = []
    for w in weights:
        nd = w.ndim
        w_specs.append(pl.BlockSpec(
            (1,) + w.shape[1:],
            lambda b, gid_ref, _nd=nd: (gid_ref[b],) + (0,) * (_nd - 1)))
    grid_spec = pltpu.PrefetchScalarGridSpec(
        num_scalar_prefetch=1,
        grid=(NB,),
        in_specs=[pl.BlockSpec((BLK, x.shape[1]), lambda b, gid_ref: (b, 0))]
        + w_specs,
        out_specs=pl.BlockSpec((BLK, D), lambda b, gid_ref: (b, 0)),
    )
    return pl.pallas_call(
        body, grid_spec=grid_spec,
        out_shape=jax.ShapeDtypeStruct((P, D), jnp.float32),
    )(block_gid, x, *weights)


def _gmm_mlp(x, block_gid, W1, b1, W2, b2):
    def _c(h, g):
        return jnp.minimum(h, 1 + g // 4)

    grid_spec = pltpu.PrefetchScalarGridSpec(
        num_scalar_prefetch=1,
        grid=(NB, NHB),
        in_specs=[
            pl.BlockSpec((BLK, D), lambda b, h, gid_ref: (b, 0)),
            pl.BlockSpec((1, D, D),
                         lambda b, h, gid_ref: (gid_ref[b], 0,
                                                _c(h, gid_ref[b]))),
            pl.BlockSpec((1, 1, D),
                         lambda b, h, gid_ref: (gid_ref[b], 0,
                                                _c(h, gid_ref[b]))),
            pl.BlockSpec((1, D, D),
                         lambda b, h, gid_ref: (gid_ref[b],
                                                _c(h, gid_ref[b]), 0)),
            pl.BlockSpec((1, 1, D), lambda b, h, gid_ref: (gid_ref[b], 0, 0)),
        ],
        out_specs=pl.BlockSpec((BLK, D), lambda b, h, gid_ref: (b, 0)),
    )
    return pl.pallas_call(
        _mlp_body, grid_spec=grid_spec,
        out_shape=jax.ShapeDtypeStruct((P, D), jnp.float32),
    )(block_gid, x, W1, b1.reshape(NE, 1, MAX_H), W2, b2.reshape(NE, 1, D))


def _combine_body(tv_ref, y_ref, o_ref):
    w0 = tv_ref[0, :][:, None]
    w1 = tv_ref[1, :][:, None]
    o_ref[...] = w0 * y_ref[:S, :] + w1 * y_ref[S:, :]


def _combine(tv, y4):
    return pl.pallas_call(
        _combine_body,
        out_shape=jax.ShapeDtypeStruct((S, D), jnp.float32),
    )(tv, y4)


def _bucketize(key):
    onehot = (key[:, None] == jnp.arange(NE, dtype=jnp.int32)[None, :])
    cum = jnp.cumsum(onehot.astype(jnp.int32), axis=0)
    counts = cum[-1]
    rank = jnp.take_along_axis(cum, key[:, None], axis=1)[:, 0] - 1
    aligned = ((counts + BLK - 1) // BLK) * BLK
    gend = jnp.cumsum(aligned)
    gstart = gend - aligned
    dest = gstart[key] + rank
    block_gid = jnp.searchsorted(
        gend, jnp.arange(NB, dtype=jnp.int32) * BLK, side='right')
    block_gid = jnp.minimum(block_gid, NE - 1).astype(jnp.int32)
    return dest.astype(jnp.int32), block_gid


NW = 32
CH = TS // NW


def _permute(src, gidx, sidx, n_out):
    mesh = plsc.VectorSubcoreMesh(core_axis_name="c", subcore_axis_name="s")

    @functools.partial(
        pl.kernel, mesh=mesh,
        out_type=jax.ShapeDtypeStruct((n_out, D), jnp.float32),
        scratch_types=[
            pltpu.VMEM((CH,), jnp.int32),
            pltpu.VMEM((CH,), jnp.int32),
            pltpu.VMEM((CH, D), jnp.float32),
            pltpu.SemaphoreType.DMA,
            pltpu.SemaphoreType.DMA,
        ],
    )
    def k(src_hbm, gidx_hbm, sidx_hbm, out_hbm, gi_v, si_v, rows_v, sem_g,
          sem_s):
        wid = lax.axis_index("s") * 2 + lax.axis_index("c")
        base = wid * CH
        pltpu.sync_copy(gidx_hbm.at[pl.ds(base, CH)], gi_v)
        pltpu.sync_copy(sidx_hbm.at[pl.ds(base, CH)], si_v)
        pltpu.async_copy(src_hbm.at[gi_v], rows_v, sem_g).wait()
        pltpu.async_copy(rows_v, out_hbm.at[si_v], sem_s).wait()

    return k(src, gidx, sidx)


def kernel(x, temperature, W_r1, b_r1, W_r2, b_r2, W_r3, b_r3,
           W_pre, b_pre, g_pre, be_pre, W1, b1, W2, b2,
           W_post, b_post, g_post, be_post):
    Bb, Ss, d = x.shape
    x2d = x.reshape(S, D)

    tv, ti, loss = _router(x2d, temperature, W_r1, b_r1, W_r2, b_r2,
                           W_r3, b_r3)

    idx_flat = ti.reshape(TS)
    pre_key = idx_flat // (NE * NE)
    rem = idx_flat % (NE * NE)
    mlp_key = rem // NE
    post_key = rem % NE
    tok = jnp.tile(jnp.arange(S, dtype=jnp.int32), TOP_K)

    dest_pre, gid_pre = _bucketize(pre_key)
    dest_mlp, gid_mlp = _bucketize(mlp_key)
    dest_post, gid_post = _bucketize(post_key)

    x1 = _permute(x2d, tok, dest_pre, P)
    y1 = _gmm(_pre_body, x1, gid_pre, W_pre, b_pre, g_pre, be_pre)
    x2 = _permute(y1, dest_pre, dest_mlp, P)
    y2 = _gmm_mlp(x2, gid_mlp, W1, b1, W2, b2)
    x3 = _permute(y2, dest_mlp, dest_post, P)
    y3 = _gmm(_post_body, x3, gid_post, W_post, b_post, g_post, be_post)
    y4 = _permute(y3, dest_post, jnp.arange(TS, dtype=jnp.int32), TS)

    out = _combine(tv, y4)
    return out.reshape(Bb, Ss, d), loss.reshape(())


# speedup vs baseline: 1.1038x; 1.1038x over previous
import functools

import jax
import jax.numpy as jnp
from jax import lax
from jax.experimental import pallas as pl
from jax.experimental.pallas import tpu as pltpu
from jax.experimental.pallas import tpu_sc as plsc

D = 768
HID = 256
NE = 8
TOTAL = 512
TOP_K = 2
MAX_H = 3 * D
S = 2048
TS = TOP_K * S
BLK = 128
NB = (TS + NE * BLK) // BLK
P = NB * BLK
EPS = 1e-5


def _gelu(v):
    return 0.5 * v * (1.0 + lax.erf(v * 0.7071067811865476))


def _layernorm(y, g, b):
    mu = y.mean(axis=-1, keepdims=True)
    var = ((y - mu) ** 2).mean(axis=-1, keepdims=True)
    return (y - mu) / jnp.sqrt(var + EPS) * g + b


def _router_body(x_ref, t_ref, w1_ref, b1_ref, w2_ref, b2_ref, w3_ref, b3_ref,
                 tv_ref, ti_ref, loss_ref):
    x = x_ref[...]
    h = _gelu(jnp.dot(x, w1_ref[...], preferred_element_type=jnp.float32)
              + b1_ref[...])
    h = _gelu(jnp.dot(h, w2_ref[...], preferred_element_type=jnp.float32)
              + b2_ref[...])
    scores = (jnp.dot(h, w3_ref[...], preferred_element_type=jnp.float32)
              + b3_ref[...])

    p0 = jax.nn.softmax(scores, axis=-1)
    freq = p0.mean(axis=0)
    mu = freq.mean()
    loss = TOTAL * jnp.sum((freq - mu) ** 2) / (TOTAL - 1)
    loss_ref[...] = loss.reshape(1, 1)

    p = jax.nn.softmax(scores / t_ref[0, 0], axis=-1)
    iota = lax.broadcasted_iota(jnp.int32, (S, TOTAL), 1)
    m1 = p.max(axis=-1)
    i1 = jnp.where(p == m1[:, None], iota, TOTAL).min(axis=-1)
    pm = jnp.where(iota == i1[:, None], -1.0, p)
    m2 = pm.max(axis=-1)
    i2 = jnp.where(pm == m2[:, None], iota, TOTAL).min(axis=-1)
    tv_ref[0, :] = m1
    tv_ref[1, :] = m2
    ti_ref[0, :] = i1
    ti_ref[1, :] = i2


def _router(x2d, temperature, W_r1, b_r1, W_r2, b_r2, W_r3, b_r3):
    return pl.pallas_call(
        _router_body,
        out_shape=(
            jax.ShapeDtypeStruct((TOP_K, S), jnp.float32),
            jax.ShapeDtypeStruct((TOP_K, S), jnp.int32),
            jax.ShapeDtypeStruct((1, 1), jnp.float32),
        ),
    )(x2d, temperature.reshape(1, 1), W_r1, b_r1.reshape(1, HID),
      W_r2, b_r2.reshape(1, HID // 2), W_r3, b_r3.reshape(1, TOTAL))


def _pre_body(gid_ref, x_ref, w_ref, b_ref, g_ref, be_ref, o_ref):
    g = gid_ref[pl.program_id(0)]
    y = (jnp.dot(x_ref[...], w_ref[0], preferred_element_type=jnp.float32)
         + b_ref[0])
    y = _layernorm(y, g_ref[0], be_ref[0])
    r = g % 3
    y = jnp.where(r == 0, _gelu(y), jnp.where(r == 1, jax.nn.relu(y),
                                              jnp.tanh(y)))
    o_ref[...] = y


def _mlp_body(gid_ref, x_ref, w1_ref, b1_ref, w2_ref, b2_ref, o_ref):
    g = gid_ref[pl.program_id(0)]
    h = (jnp.dot(x_ref[...], w1_ref[0], preferred_element_type=jnp.float32)
         + b1_ref[0])
    h = jnp.where(g % 2 == 0, _gelu(h), jax.nn.relu(h))
    o_ref[...] = (jnp.dot(h, w2_ref[0], preferred_element_type=jnp.float32)
                  + b2_ref[0])


def _post_body(gid_ref, x_ref, w_ref, b_ref, g_ref, be_ref, o_ref):
    g = gid_ref[pl.program_id(0)]
    z = (jnp.dot(x_ref[...], w_ref[0], preferred_element_type=jnp.float32)
         + b_ref[0])
    o_ref[...] = jnp.where(g % 2 == 0, _layernorm(z, g_ref[0], be_ref[0]), z)


def _gmm(body, x, block_gid, *weights):
    weights = tuple(w.reshape(w.shape[0], 1, w.shape[1]) if w.ndim == 2 else w
                    for w in weights)
    w_specs = []
    for w in weights:
        nd = w.ndim
        w_specs.append(pl.BlockSpec(
            (1,) + w.shape[1:],
            lambda b, gid_ref, _nd=nd: (gid_ref[b],) + (0,) * (_nd - 1)))
    grid_spec = pltpu.PrefetchScalarGridSpec(
        num_scalar_prefetch=1,
        grid=(NB,),
        in_specs=[pl.BlockSpec((BLK, x.shape[1]), lambda b, gid_ref: (b, 0))]
        + w_specs,
        out_specs=pl.BlockSpec((BLK, D), lambda b, gid_ref: (b, 0)),
    )
    return pl.pallas_call(
        body, grid_spec=grid_spec,
        out_shape=jax.ShapeDtypeStruct((P, D), jnp.float32),
    )(block_gid, x, *weights)


def _combine_body(tv_ref, y_ref, o_ref):
    w0 = tv_ref[0, :][:, None]
    w1 = tv_ref[1, :][:, None]
    o_ref[...] = w0 * y_ref[:S, :] + w1 * y_ref[S:, :]


def _combine(tv, y4):
    return pl.pallas_call(
        _combine_body,
        out_shape=jax.ShapeDtypeStruct((S, D), jnp.float32),
    )(tv, y4)


def _bucketize3(keys):
    onehot = (keys[..., None] == jnp.arange(NE, dtype=jnp.int32))
    cum = jnp.cumsum(onehot.astype(jnp.int32), axis=1)
    counts = cum[:, -1]
    rank = jnp.take_along_axis(cum, keys[..., None], axis=2)[..., 0] - 1
    aligned = ((counts + BLK - 1) // BLK) * BLK
    gend = jnp.cumsum(aligned, axis=1)
    gstart = gend - aligned
    dest = jnp.take_along_axis(gstart, keys, axis=1) + rank
    block_gid = jnp.sum(
        jnp.arange(NB, dtype=jnp.int32)[None, :, None] * BLK
        >= gend[:, None, :], axis=2)
    block_gid = jnp.minimum(block_gid, NE - 1).astype(jnp.int32)
    return dest.astype(jnp.int32), block_gid


NW = 32
CH = TS // NW


def _permute(src, gidx, sidx, n_out):
    mesh = plsc.VectorSubcoreMesh(core_axis_name="c", subcore_axis_name="s")

    @functools.partial(
        pl.kernel, mesh=mesh,
        out_type=jax.ShapeDtypeStruct((n_out, D), jnp.float32),
        scratch_types=[
            pltpu.VMEM((CH,), jnp.int32),
            pltpu.VMEM((CH,), jnp.int32),
            pltpu.VMEM((CH, D), jnp.float32),
            pltpu.SemaphoreType.DMA,
            pltpu.SemaphoreType.DMA,
        ],
    )
    def k(src_hbm, gidx_hbm, sidx_hbm, out_hbm, gi_v, si_v, rows_v, sem_g,
          sem_s):
        wid = lax.axis_index("s") * 2 + lax.axis_index("c")
        base = wid * CH
        pltpu.sync_copy(gidx_hbm.at[pl.ds(base, CH)], gi_v)
        pltpu.sync_copy(sidx_hbm.at[pl.ds(base, CH)], si_v)
        pltpu.async_copy(src_hbm.at[gi_v], rows_v, sem_g).wait()
        pltpu.async_copy(rows_v, out_hbm.at[si_v], sem_s).wait()

    return k(src, gidx, sidx)


def kernel(x, temperature, W_r1, b_r1, W_r2, b_r2, W_r3, b_r3,
           W_pre, b_pre, g_pre, be_pre, W1, b1, W2, b2,
           W_post, b_post, g_post, be_post):
    Bb, Ss, d = x.shape
    x2d = x.reshape(S, D)

    tv, ti, loss = _router(x2d, temperature, W_r1, b_r1, W_r2, b_r2,
                           W_r3, b_r3)

    idx_flat = ti.reshape(TS)
    rem = idx_flat % (NE * NE)
    keys = jnp.stack([idx_flat // (NE * NE), rem // NE, rem % NE])
    tok = jnp.tile(jnp.arange(S, dtype=jnp.int32), TOP_K)

    dest3, gid3 = _bucketize3(keys)
    dest_pre, dest_mlp, dest_post = dest3[0], dest3[1], dest3[2]
    gid_pre, gid_mlp, gid_post = gid3[0], gid3[1], gid3[2]

    x1 = _permute(x2d, tok, dest_pre, P)
    y1 = _gmm(_pre_body, x1, gid_pre, W_pre, b_pre, g_pre, be_pre)
    x2 = _permute(y1, dest_pre, dest_mlp, P)
    y2 = _gmm(_mlp_body, x2, gid_mlp, W1, b1, W2, b2)
    x3 = _permute(y2, dest_mlp, dest_post, P)
    y3 = _gmm(_post_body, x3, gid_post, W_post, b_post, g_post, be_post)
    y4 = _permute(y3, dest_post, jnp.arange(TS, dtype=jnp.int32), TS)

    out = _combine(tv, y4)
    return out.reshape(Bb, Ss, d), loss.reshape(())


# speedup vs baseline: 1.2839x; 1.1632x over previous
import functools

import jax
import jax.numpy as jnp
from jax import lax
from jax.experimental import pallas as pl
from jax.experimental.pallas import tpu as pltpu
from jax.experimental.pallas import tpu_sc as plsc

D = 768
HID = 256
NE = 8
TOTAL = 512
TOP_K = 2
MAX_H = 3 * D
S = 2048
TS = TOP_K * S
BLK = 128
NB = (TS + NE * BLK) // BLK
P = NB * BLK
EPS = 1e-5


def _gelu(v):
    return 0.5 * v * (1.0 + lax.erf(v * 0.7071067811865476))


def _layernorm(y, g, b):
    mu = y.mean(axis=-1, keepdims=True)
    var = ((y - mu) ** 2).mean(axis=-1, keepdims=True)
    return (y - mu) / jnp.sqrt(var + EPS) * g + b


def _router_body(x_ref, t_ref, w1_ref, b1_ref, w2_ref, b2_ref, w3_ref, b3_ref,
                 tv_ref, ti_ref, loss_ref):
    x = x_ref[...]
    h = _gelu(jnp.dot(x, w1_ref[...], preferred_element_type=jnp.float32)
              + b1_ref[...])
    h = _gelu(jnp.dot(h, w2_ref[...], preferred_element_type=jnp.float32)
              + b2_ref[...])
    scores = (jnp.dot(h, w3_ref[...], preferred_element_type=jnp.float32)
              + b3_ref[...])

    p0 = jax.nn.softmax(scores, axis=-1)
    freq = p0.mean(axis=0)
    mu = freq.mean()
    loss = TOTAL * jnp.sum((freq - mu) ** 2) / (TOTAL - 1)
    loss_ref[...] = loss.reshape(1, 1)

    p = jax.nn.softmax(scores / t_ref[0, 0], axis=-1)
    iota = lax.broadcasted_iota(jnp.int32, (S, TOTAL), 1)
    m1 = p.max(axis=-1)
    i1 = jnp.where(p == m1[:, None], iota, TOTAL).min(axis=-1)
    pm = jnp.where(iota == i1[:, None], -1.0, p)
    m2 = pm.max(axis=-1)
    i2 = jnp.where(pm == m2[:, None], iota, TOTAL).min(axis=-1)
    tv_ref[0, :] = m1
    tv_ref[1, :] = m2
    ti_ref[0, :] = i1
    ti_ref[1, :] = i2


def _router(x2d, temperature, W_r1, b_r1, W_r2, b_r2, W_r3, b_r3):
    return pl.pallas_call(
        _router_body,
        out_shape=(
            jax.ShapeDtypeStruct((TOP_K, S), jnp.float32),
            jax.ShapeDtypeStruct((TOP_K, S), jnp.int32),
            jax.ShapeDtypeStruct((1, 1), jnp.float32),
        ),
    )(x2d, temperature.reshape(1, 1), W_r1, b_r1.reshape(1, HID),
      W_r2, b_r2.reshape(1, HID // 2), W_r3, b_r3.reshape(1, TOTAL))


def _pre_body(gid_ref, x_ref, w_ref, b_ref, g_ref, be_ref, o_ref):
    g = gid_ref[pl.program_id(0)]
    y = (jnp.dot(x_ref[...], w_ref[0], preferred_element_type=jnp.float32)
         + b_ref[0])
    y = _layernorm(y, g_ref[0], be_ref[0])
    r = g % 3
    y = jnp.where(r == 0, _gelu(y), jnp.where(r == 1, jax.nn.relu(y),
                                              jnp.tanh(y)))
    o_ref[...] = y


def _mlp_body(gid_ref, x_ref, w1_ref, b1_ref, w2_ref, b2_ref, o_ref):
    g = gid_ref[pl.program_id(0)]
    h = (jnp.dot(x_ref[...], w1_ref[0], preferred_element_type=jnp.float32)
         + b1_ref[0])
    h = jnp.where(g % 2 == 0, _gelu(h), jax.nn.relu(h))
    o_ref[...] = (jnp.dot(h, w2_ref[0], preferred_element_type=jnp.float32)
                  + b2_ref[0])


def _post_body(gid_ref, x_ref, w_ref, b_ref, g_ref, be_ref, o_ref):
    g = gid_ref[pl.program_id(0)]
    z = (jnp.dot(x_ref[...], w_ref[0], preferred_element_type=jnp.float32)
         + b_ref[0])
    o_ref[...] = jnp.where(g % 2 == 0, _layernorm(z, g_ref[0], be_ref[0]), z)


def _gmm(body, x, block_gid, *weights):
    weights = tuple(w.reshape(w.shape[0], 1, w.shape[1]) if w.ndim == 2 else w
                    for w in weights)
    w_specs = []
    for w in weights:
        nd = w.ndim
        w_specs.append(pl.BlockSpec(
            (1,) + w.shape[1:],
            lambda b, gid_ref, _nd=nd: (gid_ref[b],) + (0,) * (_nd - 1)))
    grid_spec = pltpu.PrefetchScalarGridSpec(
        num_scalar_prefetch=1,
        grid=(NB,),
        in_specs=[pl.BlockSpec((BLK, x.shape[1]), lambda b, gid_ref: (b, 0))]
        + w_specs,
        out_specs=pl.BlockSpec((BLK, D), lambda b, gid_ref: (b, 0)),
    )
    return pl.pallas_call(
        body, grid_spec=grid_spec,
        out_shape=jax.ShapeDtypeStruct((P, D), jnp.float32),
    )(block_gid, x, *weights)


def _combine_body(tv_ref, y_ref, o_ref):
    w0 = tv_ref[0, :][:, None]
    w1 = tv_ref[1, :][:, None]
    o_ref[...] = w0 * y_ref[:S, :] + w1 * y_ref[S:, :]


def _combine(tv, y4):
    return pl.pallas_call(
        _combine_body,
        out_shape=jax.ShapeDtypeStruct((S, D), jnp.float32),
    )(tv, y4)


def _bucketize(key):
    onehot = (key[:, None] == jnp.arange(NE, dtype=jnp.int32)[None, :])
    cum = jnp.cumsum(onehot.astype(jnp.int32), axis=0)
    counts = cum[-1]
    rank = jnp.take_along_axis(cum, key[:, None], axis=1)[:, 0] - 1
    aligned = ((counts + BLK - 1) // BLK) * BLK
    gend = jnp.cumsum(aligned)
    gstart = gend - aligned
    dest = gstart[key] + rank
    block_gid = jnp.searchsorted(
        gend, jnp.arange(NB, dtype=jnp.int32) * BLK, side='right')
    block_gid = jnp.minimum(block_gid, NE - 1).astype(jnp.int32)
    return dest.astype(jnp.int32), block_gid


NW = 32
CH = TS // NW


def _permute(src, gidx, sidx, n_out):
    mesh = plsc.VectorSubcoreMesh(core_axis_name="c", subcore_axis_name="s")

    @functools.partial(
        pl.kernel, mesh=mesh,
        out_type=jax.ShapeDtypeStruct((n_out, D), jnp.float32),
        scratch_types=[
            pltpu.VMEM((CH,), jnp.int32),
            pltpu.VMEM((CH,), jnp.int32),
            pltpu.VMEM((CH, D), jnp.float32),
            pltpu.SemaphoreType.DMA,
            pltpu.SemaphoreType.DMA,
        ],
    )
    def k(src_hbm, gidx_hbm, sidx_hbm, out_hbm, gi_v, si_v, rows_v, sem_g,
          sem_s):
        wid = lax.axis_index("s") * 2 + lax.axis_index("c")
        base = wid * CH
        pltpu.sync_copy(gidx_hbm.at[pl.ds(base, CH)], gi_v)
        pltpu.sync_copy(sidx_hbm.at[pl.ds(base, CH)], si_v)
        pltpu.async_copy(src_hbm.at[gi_v], rows_v, sem_g).wait()
        pltpu.async_copy(rows_v, out_hbm.at[si_v], sem_s).wait()

    return k(src, gidx, sidx)


def kernel(x, temperature, W_r1, b_r1, W_r2, b_r2, W_r3, b_r3,
           W_pre, b_pre, g_pre, be_pre, W1, b1, W2, b2,
           W_post, b_post, g_post, be_post):
    Bb, Ss, d = x.shape
    x2d = x.reshape(S, D)

    tv, ti, loss = _router(x2d, temperature, W_r1, b_r1, W_r2, b_r2,
                           W_r3, b_r3)

    idx_flat = ti.reshape(TS)
    pre_key = idx_flat // (NE * NE)
    rem = idx_flat % (NE * NE)
    mlp_key = rem // NE
    post_key = rem % NE
    tok = jnp.tile(jnp.arange(S, dtype=jnp.int32), TOP_K)

    dest_pre, gid_pre = _bucketize(pre_key)
    dest_mlp, gid_mlp = _bucketize(mlp_key)
    dest_post, gid_post = _bucketize(post_key)

    x1 = _permute(x2d, tok, dest_pre, P)
    y1 = _gmm(_pre_body, x1, gid_pre, W_pre, b_pre, g_pre, be_pre)
    x2 = _permute(y1, dest_pre, dest_mlp, P)
    y2 = _gmm(_mlp_body, x2, gid_mlp, W1, b1, W2, b2)
    x3 = _permute(y2, dest_mlp, dest_post, P)
    y3 = _gmm(_post_body, x3, gid_post, W_post, b_post, g_post, be_post)
    y4 = _permute(y3, dest_post, jnp.arange(TS, dtype=jnp.int32), TS)

    out = _combine(tv, y4)
    return out.reshape(Bb, Ss, d), loss.reshape(())
